# baseline (speedup 1.0000x reference)
import functools

import jax
import jax.numpy as jnp
from jax import lax
from jax.experimental import pallas as pl
from jax.experimental.pallas import tpu as pltpu
from jax.experimental.pallas import tpu_sc as plsc

_VOCAB = 1000000
_EMBED = 64
_OUT = 3
_DPAD = 8
_BATCH = 16384
_HIST = 50
_NIDX = _BATCH * _HIST

_NC, _NS = 2, 16
_NW = _NC * _NS
_BPW = _NIDX // _NW
_CHUNK = 3200
_NCHUNK = _BPW // _CHUNK

_MM_COLS = 32768


def _mm_body(t_ref, w_ref, b_ref, o_ref):
    o_ref[...] = lax.dot_general(
        w_ref[...], t_ref[...], (((0,), (0,)), ((), ())),
        preferred_element_type=jnp.float32,
    ) + b_ref[...]


def _project_table(emb_t, w_pad, b_pad):
    grid = (_VOCAB + _MM_COLS - 1) // _MM_COLS
    return pl.pallas_call(
        _mm_body,
        grid=(grid,),
        in_specs=[
            pl.BlockSpec((_EMBED, _MM_COLS), lambda i: (0, i)),
            pl.BlockSpec((_EMBED, _DPAD), lambda i: (0, 0)),
            pl.BlockSpec((_DPAD, 1), lambda i: (0, 0)),
        ],
        out_specs=pl.BlockSpec((_DPAD, _MM_COLS), lambda i: (0, i)),
        out_shape=jax.ShapeDtypeStruct((_DPAD, _VOCAB), jnp.float32),
    )(emb_t, w_pad, b_pad)


_sc_mesh = plsc.VectorSubcoreMesh(core_axis_name="c", subcore_axis_name="s")


_BUILD_UNROLL = 8


@functools.partial(
    pl.kernel,
    mesh=_sc_mesh,
    compiler_params=pltpu.CompilerParams(
        use_tc_tiling_on_sc=False, needs_layout_passes=False
    ),
    out_type=jax.ShapeDtypeStruct((_OUT, _NIDX), jnp.float32),
    scratch_types=[
        pltpu.VMEM((2, _CHUNK), jnp.int32),
        pltpu.VMEM((2, 2 * _CHUNK), jnp.int32),
        pltpu.VMEM((2, _CHUNK), jnp.float32),
        pltpu.VMEM((2, 2 * _CHUNK), jnp.float32),
        pltpu.SemaphoreType.DMA,
        pltpu.SemaphoreType.DMA,
    ],
)
def _gather_sc(proj_hbm, idx_hbm, out_hbm, idx_v, cidx_v, v0_v, v12_v, s0, s1):
    wid = lax.axis_index("s") * _NC + lax.axis_index("c")
    base = wid * _BPW
    sems = (s0, s1)

    def _start(ci):
        b = ci % 2
        off = base + ci * _CHUNK
        pltpu.sync_copy(idx_hbm.at[pl.ds(off, _CHUNK)], idx_v.at[b])

        def _bld(t, _):
            for u in range(_BUILD_UNROLL):
                s = (t * _BUILD_UNROLL + u) * 16
                v = idx_v[b, pl.ds(s, 16)]
                cidx_v[b, pl.ds(s, 16)] = v + _VOCAB
                cidx_v[b, pl.ds(_CHUNK + s, 16)] = v + 2 * _VOCAB
            return _

        lax.fori_loop(0, _CHUNK // (16 * _BUILD_UNROLL), _bld, 0)
        c0 = pltpu.async_copy(proj_hbm.at[idx_v.at[b]], v0_v.at[b], sems[b])
        c12 = pltpu.async_copy(proj_hbm.at[cidx_v.at[b]], v12_v.at[b], sems[b])
        return c0, c12

    def _finish(ci, c0, c12):
        b = ci % 2
        off = base + ci * _CHUNK
        c0.wait()
        c12.wait()
        pltpu.sync_copy(v0_v.at[b], out_hbm.at[0, pl.ds(off, _CHUNK)])
        for c in (1, 2):
            pltpu.sync_copy(
                v12_v.at[b, pl.ds((c - 1) * _CHUNK, _CHUNK)],
                out_hbm.at[c, pl.ds(off, _CHUNK)],
            )

    pending = _start(0)
    for ci in range(1, _NCHUNK):
        nxt = _start(ci)
        _finish(ci - 1, *pending)
        pending = nxt
    _finish(_NCHUNK - 1, *pending)


def kernel(x, emb_table, W, b):
    w_pad = jnp.zeros((_EMBED, _DPAD), jnp.float32).at[:, :_OUT].set(W)
    b_pad = jnp.zeros((_DPAD, 1), jnp.float32).at[:_OUT, 0].set(b)
    proj_t = _project_table(emb_table.T, w_pad, b_pad)
    idx_lin = x.T.reshape(-1)
    out2 = _gather_sc(proj_t.reshape(-1), idx_lin)
    return out2.reshape(_OUT, _HIST, _BATCH).transpose(2, 1, 0)


# speedup vs baseline: 2.4210x; 2.4210x over previous
import functools

import jax
import jax.numpy as jnp
from jax import lax
from jax.experimental import pallas as pl
from jax.experimental.pallas import tpu as pltpu
from jax.experimental.pallas import tpu_sc as plsc

_VOCAB = 1000000
_EMBED = 64
_OUT = 3
_DPAD = 8
_BATCH = 16384
_HIST = 50
_NIDX = _BATCH * _HIST

_NC, _NS = 2, 16
_NW = _NC * _NS
_BPW = _NIDX // _NW
_CHUNK = 3200
_NCHUNK = _BPW // _CHUNK

_MM_COLS = 32768


def _mm_body(t_ref, w_ref, b_ref, o0_ref, o1_ref, o2_ref):
    t = t_ref[...]
    for c, o_ref in enumerate((o0_ref, o1_ref, o2_ref)):
        o_ref[...] = lax.dot_general(
            w_ref[:, c:c + 1], t, (((0,), (0,)), ((), ())),
            preferred_element_type=jnp.float32,
        ) + b_ref[c, 0]


def _project_table(emb_t, w, b2):
    grid = (_VOCAB + _MM_COLS - 1) // _MM_COLS
    plane = jax.ShapeDtypeStruct((1, _VOCAB), jnp.float32)
    return pl.pallas_call(
        _mm_body,
        grid=(grid,),
        in_specs=[
            pl.BlockSpec((_EMBED, _MM_COLS), lambda i: (0, i)),
            pl.BlockSpec((_EMBED, _OUT), lambda i: (0, 0)),
            pl.BlockSpec((_OUT, 1), lambda i: (0, 0)),
        ],
        out_specs=[pl.BlockSpec((1, _MM_COLS), lambda i: (0, i))] * _OUT,
        out_shape=[plane] * _OUT,
    )(emb_t, w, b2)


_sc_mesh = plsc.VectorSubcoreMesh(core_axis_name="c", subcore_axis_name="s")


@functools.partial(
    pl.kernel,
    mesh=_sc_mesh,
    compiler_params=pltpu.CompilerParams(
        use_tc_tiling_on_sc=False, needs_layout_passes=False
    ),
    out_type=jax.ShapeDtypeStruct((_OUT, _NIDX), jnp.float32),
    scratch_types=[
        pltpu.VMEM((2, _CHUNK), jnp.int32),
        pltpu.VMEM((2, _OUT, _CHUNK), jnp.float32),
        pltpu.SemaphoreType.DMA,
        pltpu.SemaphoreType.DMA,
    ],
)
def _gather_sc(p0, p1, p2, idx_hbm, out_hbm, idx_v, vals_v, s0, s1):
    wid = lax.axis_index("s") * _NC + lax.axis_index("c")
    base = wid * _BPW
    planes = (p0, p1, p2)
    sems = (s0, s1)

    def _start(ci):
        b = ci % 2
        off = base + ci * _CHUNK
        pltpu.sync_copy(idx_hbm.at[pl.ds(off, _CHUNK)], idx_v.at[b])
        return [
            pltpu.async_copy(planes[c].at[idx_v.at[b]], vals_v.at[b, c], sems[b])
            for c in range(_OUT)
        ]

    def _finish(ci, copies):
        b = ci % 2
        off = base + ci * _CHUNK
        for cp in copies:
            cp.wait()
        for c in range(_OUT):
            pltpu.sync_copy(vals_v.at[b, c], out_hbm.at[c, pl.ds(off, _CHUNK)])

    pending = _start(0)
    for ci in range(1, _NCHUNK):
        nxt = _start(ci)
        _finish(ci - 1, pending)
        pending = nxt
    _finish(_NCHUNK - 1, pending)


def kernel(x, emb_table, W, b):
    p0, p1, p2 = _project_table(emb_table.T, W, b.reshape(_OUT, 1))
    idx_lin = x.T.reshape(-1)
    out2 = _gather_sc(p0.reshape(-1), p1.reshape(-1), p2.reshape(-1), idx_lin)
    return out2.reshape(_OUT, _HIST, _BATCH).transpose(2, 1, 0)


# speedup vs baseline: 3.8626x; 1.5955x over previous
import functools

import jax
import jax.numpy as jnp
from jax import lax
from jax.experimental import pallas as pl
from jax.experimental.pallas import tpu as pltpu
from jax.experimental.pallas import tpu_sc as plsc

_VOCAB = 1000000
_EMBED = 64
_OUT = 3
_DPAD = 8
_BATCH = 16384
_HIST = 50
_NIDX = _BATCH * _HIST

_NC, _NS = 2, 16
_NW = _NC * _NS
_BPW = _NIDX // _NW
_CHUNK = 3200
_NCHUNK = _BPW // _CHUNK

_MM_COLS = 32768


def _mm_body(t_ref, w_ref, b_ref, o0_ref, o1_ref, o2_ref):
    t = t_ref[...]
    for c, o_ref in enumerate((o0_ref, o1_ref, o2_ref)):
        p = lax.dot_general(
            w_ref[:, c:c + 1], t, (((0,), (0,)), ((), ())),
            preferred_element_type=jnp.float32,
        ) + b_ref[c, 0]
        o_ref[...] = p.reshape(_MM_COLS)


def _project_table(emb_t, w, b2):
    grid = (_VOCAB + _MM_COLS - 1) // _MM_COLS
    plane = jax.ShapeDtypeStruct((_VOCAB,), jnp.float32)
    return pl.pallas_call(
        _mm_body,
        grid=(grid,),
        in_specs=[
            pl.BlockSpec((_EMBED, _MM_COLS), lambda i: (0, i)),
            pl.BlockSpec((_EMBED, _OUT), lambda i: (0, 0)),
            pl.BlockSpec((_OUT, 1), lambda i: (0, 0)),
        ],
        out_specs=[pl.BlockSpec((_MM_COLS,), lambda i: (i,))] * _OUT,
        out_shape=[plane] * _OUT,
    )(emb_t, w, b2)


_sc_mesh = plsc.VectorSubcoreMesh(core_axis_name="c", subcore_axis_name="s")


@functools.partial(
    pl.kernel,
    mesh=_sc_mesh,
    compiler_params=pltpu.CompilerParams(
        use_tc_tiling_on_sc=False, needs_layout_passes=False
    ),
    out_type=jax.ShapeDtypeStruct((_OUT, _NIDX), jnp.float32),
    scratch_types=[
        pltpu.VMEM((2, _CHUNK), jnp.int32),
        pltpu.VMEM((2, _OUT, _CHUNK), jnp.float32),
        pltpu.SemaphoreType.DMA,
        pltpu.SemaphoreType.DMA,
    ],
)
def _gather_sc(p0, p1, p2, idx_hbm, out_hbm, idx_v, vals_v, s0, s1):
    wid = lax.axis_index("s") * _NC + lax.axis_index("c")
    base = wid * _BPW
    planes = (p0, p1, p2)
    sems = (s0, s1)

    def _start(ci):
        b = ci % 2
        off = base + ci * _CHUNK
        pltpu.sync_copy(idx_hbm.at[pl.ds(off, _CHUNK)], idx_v.at[b])
        return [
            pltpu.async_copy(planes[c].at[idx_v.at[b]], vals_v.at[b, c], sems[b])
            for c in range(_OUT)
        ]

    def _finish(ci, copies):
        b = ci % 2
        off = base + ci * _CHUNK
        for cp in copies:
            cp.wait()
        for c in range(_OUT):
            pltpu.sync_copy(vals_v.at[b, c], out_hbm.at[c, pl.ds(off, _CHUNK)])

    pending = _start(0)
    for ci in range(1, _NCHUNK):
        nxt = _start(ci)
        _finish(ci - 1, pending)
        pending = nxt
    _finish(_NCHUNK - 1, pending)


def kernel(x, emb_table, W, b):
    p0, p1, p2 = _project_table(emb_table.T, W, b.reshape(_OUT, 1))
    idx_lin = x.T.reshape(-1)
    out2 = _gather_sc(p0, p1, p2, idx_lin)
    return out2.reshape(_OUT, _HIST, _BATCH).transpose(2, 1, 0)


# speedup vs baseline: 3.8737x; 1.0028x over previous
import functools

import jax
import jax.numpy as jnp
from jax import lax
from jax.experimental import pallas as pl
from jax.experimental.pallas import tpu as pltpu
from jax.experimental.pallas import tpu_sc as plsc

_VOCAB = 1000000
_EMBED = 64
_OUT = 3
_DPAD = 8
_BATCH = 16384
_HIST = 50
_NIDX = _BATCH * _HIST

_NC, _NS = 2, 16
_NW = _NC * _NS
_BPW = _NIDX // _NW
_CHUNK = 6400
_NCHUNK = _BPW // _CHUNK

_MM_COLS = 32768


def _mm_body(t_ref, w_ref, b_ref, o0_ref, o1_ref, o2_ref):
    t = t_ref[...]
    for c, o_ref in enumerate((o0_ref, o1_ref, o2_ref)):
        p = lax.dot_general(
            w_ref[:, c:c + 1], t, (((0,), (0,)), ((), ())),
            preferred_element_type=jnp.float32,
        ) + b_ref[c, 0]
        o_ref[...] = p.reshape(_MM_COLS)


def _project_table(emb_t, w, b2):
    grid = (_VOCAB + _MM_COLS - 1) // _MM_COLS
    plane = jax.ShapeDtypeStruct((_VOCAB,), jnp.float32)
    return pl.pallas_call(
        _mm_body,
        grid=(grid,),
        in_specs=[
            pl.BlockSpec((_EMBED, _MM_COLS), lambda i: (0, i)),
            pl.BlockSpec((_EMBED, _OUT), lambda i: (0, 0)),
            pl.BlockSpec((_OUT, 1), lambda i: (0, 0)),
        ],
        out_specs=[pl.BlockSpec((_MM_COLS,), lambda i: (i,))] * _OUT,
        out_shape=[plane] * _OUT,
    )(emb_t, w, b2)


_sc_mesh = plsc.VectorSubcoreMesh(core_axis_name="c", subcore_axis_name="s")


@functools.partial(
    pl.kernel,
    mesh=_sc_mesh,
    compiler_params=pltpu.CompilerParams(
        use_tc_tiling_on_sc=False, needs_layout_passes=False
    ),
    out_type=jax.ShapeDtypeStruct((_OUT, _NIDX), jnp.float32),
    scratch_types=[
        pltpu.VMEM((2, _CHUNK), jnp.int32),
        pltpu.VMEM((2, _OUT, _CHUNK), jnp.float32),
        pltpu.SemaphoreType.DMA,
        pltpu.SemaphoreType.DMA,
    ],
)
def _gather_sc(p0, p1, p2, idx_hbm, out_hbm, idx_v, vals_v, s0, s1):
    wid = lax.axis_index("s") * _NC + lax.axis_index("c")
    base = wid * _BPW
    planes = (p0, p1, p2)
    sems = (s0, s1)

    def _start(ci):
        b = ci % 2
        off = base + ci * _CHUNK
        pltpu.sync_copy(idx_hbm.at[pl.ds(off, _CHUNK)], idx_v.at[b])
        return [
            pltpu.async_copy(planes[c].at[idx_v.at[b]], vals_v.at[b, c], sems[b])
            for c in range(_OUT)
        ]

    def _finish(ci, copies):
        b = ci % 2
        off = base + ci * _CHUNK
        for cp in copies:
            cp.wait()
        for c in range(_OUT):
            pltpu.sync_copy(vals_v.at[b, c], out_hbm.at[c, pl.ds(off, _CHUNK)])

    pending = _start(0)
    for ci in range(1, _NCHUNK):
        nxt = _start(ci)
        _finish(ci - 1, pending)
        pending = nxt
    _finish(_NCHUNK - 1, pending)


def kernel(x, emb_table, W, b):
    p0, p1, p2 = _project_table(emb_table.T, W, b.reshape(_OUT, 1))
    idx_lin = x.T.reshape(-1)
    out2 = _gather_sc(p0, p1, p2, idx_lin)
    return out2.reshape(_OUT, _HIST, _BATCH).transpose(2, 1, 0)


# speedup vs baseline: 3.9452x; 1.0185x over previous
import functools

import jax
import jax.numpy as jnp
from jax import lax
from jax.experimental import pallas as pl
from jax.experimental.pallas import tpu as pltpu
from jax.experimental.pallas import tpu_sc as plsc

_VOCAB = 1000000
_EMBED = 64
_OUT = 3
_DPAD = 8
_BATCH = 16384
_HIST = 50
_NIDX = _BATCH * _HIST

_NC, _NS = 2, 16
_NW = _NC * _NS
_BPW = _NIDX // _NW
_CHUNK = 6400
_NCHUNK = _BPW // _CHUNK

_MM_COLS = 65536


def _mm_body(t_ref, w_ref, b_ref, o0_ref, o1_ref, o2_ref):
    t = t_ref[...]
    for c, o_ref in enumerate((o0_ref, o1_ref, o2_ref)):
        p = lax.dot_general(
            w_ref[:, c:c + 1], t, (((0,), (0,)), ((), ())),
            preferred_element_type=jnp.float32,
        ) + b_ref[c, 0]
        o_ref[...] = p.reshape(_MM_COLS)


def _project_table(emb_t, w, b2):
    grid = (_VOCAB + _MM_COLS - 1) // _MM_COLS
    plane = jax.ShapeDtypeStruct((_VOCAB,), jnp.float32)
    return pl.pallas_call(
        _mm_body,
        grid=(grid,),
        in_specs=[
            pl.BlockSpec((_EMBED, _MM_COLS), lambda i: (0, i)),
            pl.BlockSpec((_EMBED, _OUT), lambda i: (0, 0)),
            pl.BlockSpec((_OUT, 1), lambda i: (0, 0)),
        ],
        out_specs=[pl.BlockSpec((_MM_COLS,), lambda i: (i,))] * _OUT,
        out_shape=[plane] * _OUT,
    )(emb_t, w, b2)


_sc_mesh = plsc.VectorSubcoreMesh(core_axis_name="c", subcore_axis_name="s")


@functools.partial(
    pl.kernel,
    mesh=_sc_mesh,
    compiler_params=pltpu.CompilerParams(
        use_tc_tiling_on_sc=False, needs_layout_passes=False
    ),
    out_type=jax.ShapeDtypeStruct((_OUT, _NIDX), jnp.float32),
    scratch_types=[
        pltpu.VMEM((2, _CHUNK), jnp.int32),
        pltpu.VMEM((2, _OUT, _CHUNK), jnp.float32),
        pltpu.SemaphoreType.DMA,
        pltpu.SemaphoreType.DMA,
    ],
)
def _gather_sc(p0, p1, p2, idx_hbm, out_hbm, idx_v, vals_v, s0, s1):
    wid = lax.axis_index("s") * _NC + lax.axis_index("c")
    base = wid * _BPW
    planes = (p0, p1, p2)
    sems = (s0, s1)

    def _start(ci):
        b = ci % 2
        off = base + ci * _CHUNK
        pltpu.sync_copy(idx_hbm.at[pl.ds(off, _CHUNK)], idx_v.at[b])
        return [
            pltpu.async_copy(planes[c].at[idx_v.at[b]], vals_v.at[b, c], sems[b])
            for c in range(_OUT)
        ]

    def _finish(ci, copies):
        b = ci % 2
        off = base + ci * _CHUNK
        for cp in copies:
            cp.wait()
        for c in range(_OUT):
            pltpu.sync_copy(vals_v.at[b, c], out_hbm.at[c, pl.ds(off, _CHUNK)])

    pending = _start(0)
    for ci in range(1, _NCHUNK):
        nxt = _start(ci)
        _finish(ci - 1, pending)
        pending = nxt
    _finish(_NCHUNK - 1, pending)


def kernel(x, emb_table, W, b):
    p0, p1, p2 = _project_table(emb_table.T, W, b.reshape(_OUT, 1))
    idx_lin = x.T.reshape(-1)
    out2 = _gather_sc(p0, p1, p2, idx_lin)
    return out2.reshape(_OUT, _HIST, _BATCH).transpose(2, 1, 0)
